# baseline (speedup 1.0000x reference)
import jax
import jax.numpy as jnp
from jax.experimental import pallas as pl


_BS = 512


def _add_kernel(x_ref, pe_ref, o_ref):
    pev = pe_ref[...]
    for b in range(x_ref.shape[1]):
        o_ref[:, b, :] = x_ref[:, b, :] + pev


def kernel(x, pos_emb):
    S, B, D = x.shape
    pe = pos_emb[:S]
    return pl.pallas_call(
        _add_kernel,
        grid=(S // _BS,),
        in_specs=[
            pl.BlockSpec((_BS, B, D), lambda i: (i, 0, 0)),
            pl.BlockSpec((_BS, D), lambda i: (i, 0)),
        ],
        out_specs=pl.BlockSpec((_BS, B, D), lambda i: (i, 0, 0)),
        out_shape=jax.ShapeDtypeStruct((S, B, D), x.dtype),
    )(x, pe)


# speedup vs baseline: 1.0046x; 1.0046x over previous
import jax
from jax.experimental import pallas as pl


_BS = 512


def _add_kernel(x_ref, pe_ref, o_ref):
    pev = pe_ref[...]
    for b in range(x_ref.shape[1]):
        o_ref[:, b, :] = x_ref[:, b, :] + pev


def kernel(x, pos_emb):
    S, B, D = x.shape
    pe = pos_emb[:S]
    return pl.pallas_call(
        _add_kernel,
        grid=(S // _BS,),
        in_specs=[
            pl.BlockSpec((_BS, B, D), lambda i: (i, 0, 0)),
            pl.BlockSpec((_BS, D), lambda i: (i, 0)),
        ],
        out_specs=pl.BlockSpec((_BS, B, D), lambda i: (i, 0, 0)),
        out_shape=jax.ShapeDtypeStruct((S, B, D), x.dtype),
    )(x, pe)
